# baseline (device time: 46715 ns/iter reference)
import jax
import jax.numpy as jnp
from jax import lax
from jax.experimental import pallas as pl
from jax.experimental.pallas import tpu as pltpu

N_DEV = 4


def _gelu(y):
    c = 0.7978845608028654
    return 0.5 * y * (1.0 + jnp.tanh(c * (y + 0.044715 * y * y * y)))


def kernel(x, w_mat):
    m, k_per = x.shape
    _, n = w_mat.shape
    chunk = m // N_DEV

    def body(x_ref, w_ref, out_ref, y_ref, comm_ref, send_sems, recv_sems):
        my = lax.axis_index("i")
        left = lax.rem(my + (N_DEV - 1), N_DEV)
        right = lax.rem(my + 1, N_DEV)

        barrier_sem = pltpu.get_barrier_semaphore()
        for nbr in (left, right):
            pl.semaphore_signal(
                barrier_sem, inc=1,
                device_id=(nbr,), device_id_type=pl.DeviceIdType.MESH,
            )
        pl.semaphore_wait(barrier_sem, 2)

        y_ref[...] = jnp.dot(
            x_ref[...], w_ref[...], preferred_element_type=jnp.float32
        )

        def partial(c):
            return y_ref[pl.ds(c * chunk, chunk), :]

        c0 = lax.rem(my + (N_DEV - 1), N_DEV)
        rdma0 = pltpu.make_async_remote_copy(
            src_ref=y_ref.at[pl.ds(c0 * chunk, chunk), :],
            dst_ref=comm_ref.at[0],
            send_sem=send_sems.at[0],
            recv_sem=recv_sems.at[0],
            device_id=(right,),
            device_id_type=pl.DeviceIdType.MESH,
        )
        rdma0.start()
        rdma0.wait()

        for s in (1, 2):
            c = lax.rem(my + (N_DEV - 1 - s), N_DEV)
            comm_ref[s - 1] = comm_ref[s - 1] + partial(c)
            rdma = pltpu.make_async_remote_copy(
                src_ref=comm_ref.at[s - 1],
                dst_ref=comm_ref.at[s],
                send_sem=send_sems.at[s],
                recv_sem=recv_sems.at[s],
                device_id=(right,),
                device_id_type=pl.DeviceIdType.MESH,
            )
            rdma.start()
            rdma.wait()

        out_ref[...] = _gelu(comm_ref[N_DEV - 2] + partial(my))

    return pl.pallas_call(
        body,
        out_shape=jax.ShapeDtypeStruct((chunk, n), jnp.float32),
        in_specs=[
            pl.BlockSpec(memory_space=pltpu.VMEM),
            pl.BlockSpec(memory_space=pltpu.VMEM),
        ],
        out_specs=pl.BlockSpec(memory_space=pltpu.VMEM),
        scratch_shapes=[
            pltpu.VMEM((m, n), jnp.float32),
            pltpu.VMEM((N_DEV - 1, chunk, n), jnp.float32),
            pltpu.SemaphoreType.DMA((N_DEV - 1,)),
            pltpu.SemaphoreType.DMA((N_DEV - 1,)),
        ],
        compiler_params=pltpu.CompilerParams(collective_id=0),
    )(x, w_mat)


# device time: 26287 ns/iter; 1.7771x vs baseline; 1.7771x over previous
import jax
import jax.numpy as jnp
from jax import lax
from jax.experimental import pallas as pl
from jax.experimental.pallas import tpu as pltpu

N_DEV = 4
N_HOP = N_DEV - 1
NSEG = 2


def _gelu(y):
    c = 0.7978845608028654
    return 0.5 * y * (1.0 + jnp.tanh(c * (y + 0.044715 * y * y * y)))


def kernel(x, w_mat):
    m, k_per = x.shape
    _, n = w_mat.shape
    chunk = m // N_DEV
    n2 = n // 2
    segw = n2 // NSEG

    def body(x_ref, w_ref, out_ref, y_ref, cw_ref, ccw_ref,
             cw_send, cw_recv, ccw_send, ccw_recv):
        my = lax.axis_index("i")
        left = lax.rem(my + (N_DEV - 1), N_DEV)
        right = lax.rem(my + 1, N_DEV)

        barrier_sem = pltpu.get_barrier_semaphore()
        for nbr in (left, right):
            pl.semaphore_signal(
                barrier_sem, inc=1,
                device_id=(nbr,), device_id_type=pl.DeviceIdType.MESH,
            )
        pl.semaphore_wait(barrier_sem, 2)

        def rows(k):
            c = lax.rem(my + k, N_DEV)
            return pl.ds(c * chunk, chunk)

        def compute_chunk(k):
            y_ref[rows(k), :] = jnp.dot(
                x_ref[rows(k), :], w_ref[...],
                preferred_element_type=jnp.float32,
            )

        dirs = {
            "cw": (cw_ref, cw_send, cw_recv, n2, right),
            "ccw": (ccw_ref, ccw_send, ccw_recv, 0, left),
        }
        rdmas = {"cw": {}, "ccw": {}}

        def start_hop0(d, k, g):
            buf, ssem, rsem, off, tgt = dirs[d]
            r = pltpu.make_async_remote_copy(
                src_ref=y_ref.at[rows(k), pl.ds(off + g * segw, segw)],
                dst_ref=buf.at[0, :, pl.ds(g * segw, segw)],
                send_sem=ssem.at[0, g],
                recv_sem=rsem.at[0, g],
                device_id=(tgt,), device_id_type=pl.DeviceIdType.MESH,
            )
            r.start()
            rdmas[d][(0, g)] = r

        def acc_and_forward(d, s, k, g):
            buf, ssem, rsem, off, tgt = dirs[d]
            rdmas[d][(s - 1, g)].wait_recv()
            gsl = pl.ds(g * segw, segw)
            buf[s - 1, :, gsl] = (
                buf[s - 1, :, gsl] + y_ref[rows(k), pl.ds(off + g * segw, segw)]
            )
            r = pltpu.make_async_remote_copy(
                src_ref=buf.at[s - 1, :, gsl],
                dst_ref=buf.at[s, :, gsl],
                send_sem=ssem.at[s, g],
                recv_sem=rsem.at[s, g],
                device_id=(tgt,), device_id_type=pl.DeviceIdType.MESH,
            )
            r.start()
            rdmas[d][(s, g)] = r

        compute_chunk(N_DEV - 1)
        compute_chunk(1)
        for g in range(NSEG):
            start_hop0("cw", N_DEV - 1, g)
            start_hop0("ccw", 1, g)
        compute_chunk(2)
        compute_chunk(0)

        for s in range(1, N_HOP):
            for g in range(NSEG):
                acc_and_forward("cw", s, N_DEV - 1 - s, g)
                acc_and_forward("ccw", s, 1 + s, g)

        for g in range(NSEG):
            rdmas["cw"][(N_HOP - 1, g)].wait_recv()
            rdmas["ccw"][(N_HOP - 1, g)].wait_recv()
        out_ref[:, 0:n2] = _gelu(ccw_ref[N_HOP - 1] + y_ref[rows(0), 0:n2])
        out_ref[:, n2:n] = _gelu(cw_ref[N_HOP - 1] + y_ref[rows(0), n2:n])

        for d in ("cw", "ccw"):
            for s in range(N_HOP):
                for g in range(NSEG):
                    rdmas[d][(s, g)].wait_send()

    return pl.pallas_call(
        body,
        out_shape=jax.ShapeDtypeStruct((chunk, n), jnp.float32),
        in_specs=[
            pl.BlockSpec(memory_space=pltpu.VMEM),
            pl.BlockSpec(memory_space=pltpu.VMEM),
        ],
        out_specs=pl.BlockSpec(memory_space=pltpu.VMEM),
        scratch_shapes=[
            pltpu.VMEM((m, n), jnp.float32),
            pltpu.VMEM((N_HOP, chunk, n2), jnp.float32),
            pltpu.VMEM((N_HOP, chunk, n2), jnp.float32),
            pltpu.SemaphoreType.DMA((N_HOP, NSEG)),
            pltpu.SemaphoreType.DMA((N_HOP, NSEG)),
            pltpu.SemaphoreType.DMA((N_HOP, NSEG)),
            pltpu.SemaphoreType.DMA((N_HOP, NSEG)),
        ],
        compiler_params=pltpu.CompilerParams(collective_id=0),
    )(x, w_mat)


# device time: 26027 ns/iter; 1.7949x vs baseline; 1.0100x over previous
import jax
import jax.numpy as jnp
from jax import lax
from jax.experimental import pallas as pl
from jax.experimental.pallas import tpu as pltpu

N_DEV = 4


def _gelu(y):
    c = 0.7978845608028654
    return 0.5 * y * (1.0 + jnp.tanh(c * (y + 0.044715 * y * y * y)))


def kernel(x, w_mat):
    m, k_per = x.shape
    _, n = w_mat.shape
    chunk = m // N_DEV
    n2 = n // 2

    def body(x_ref, w_ref, out_ref, y_ref, xbuf, ybuf,
             x_send, x_recv, y_send, y_recv):
        my = lax.axis_index("i")
        c_yp = lax.bitwise_xor(my, 1)
        c_xp = 3 - my
        c_diag = lax.rem(my + 2, N_DEV)

        barrier_sem = pltpu.get_barrier_semaphore()
        for nbr in (c_yp, c_xp):
            pl.semaphore_signal(
                barrier_sem, inc=1,
                device_id=(nbr,), device_id_type=pl.DeviceIdType.MESH,
            )
        pl.semaphore_wait(barrier_sem, 2)

        def rows(c):
            return pl.ds(c * chunk, chunk)

        def compute_chunk(c):
            y_ref[rows(c), :] = jnp.dot(
                x_ref[rows(c), :], w_ref[...],
                preferred_element_type=jnp.float32,
            )

        X = pl.ds(n2, n2)
        Y = pl.ds(0, n2)

        def send(src, buf, slot, ssem, rsem, tgt):
            r = pltpu.make_async_remote_copy(
                src_ref=src,
                dst_ref=buf.at[slot],
                send_sem=ssem.at[slot],
                recv_sem=rsem.at[slot],
                device_id=(tgt,), device_id_type=pl.DeviceIdType.MESH,
            )
            r.start()
            return r

        compute_chunk(c_diag)
        x_p1a = send(y_ref.at[rows(c_diag), X], xbuf, 0, x_send, x_recv, c_xp)
        y_p1a = send(y_ref.at[rows(c_diag), Y], ybuf, 0, y_send, y_recv, c_yp)
        compute_chunk(c_yp)
        y_p1b = send(y_ref.at[rows(c_yp), Y], ybuf, 1, y_send, y_recv, c_yp)
        compute_chunk(c_xp)
        x_p1b = send(y_ref.at[rows(c_xp), X], xbuf, 1, x_send, x_recv, c_xp)
        compute_chunk(my)

        x_p1a.wait_recv()
        xbuf[0] = xbuf[0] + y_ref[rows(c_yp), X]
        y_p2 = send(xbuf.at[0], ybuf, 2, y_send, y_recv, c_yp)

        y_p1a.wait_recv()
        ybuf[0] = ybuf[0] + y_ref[rows(c_xp), Y]
        x_p2 = send(ybuf.at[0], xbuf, 2, x_send, x_recv, c_xp)

        x_p1b.wait_recv()
        y_p1b.wait_recv()
        y_p2.wait_recv()
        x_p2.wait_recv()
        out_ref[:, 0:n2] = _gelu(y_ref[rows(my), Y] + ybuf[1] + xbuf[2])
        out_ref[:, n2:n] = _gelu(y_ref[rows(my), X] + xbuf[1] + ybuf[2])

        for r in (x_p1a, y_p1a, y_p1b, x_p1b, y_p2, x_p2):
            r.wait_send()

    return pl.pallas_call(
        body,
        out_shape=jax.ShapeDtypeStruct((chunk, n), jnp.float32),
        in_specs=[
            pl.BlockSpec(memory_space=pltpu.VMEM),
            pl.BlockSpec(memory_space=pltpu.VMEM),
        ],
        out_specs=pl.BlockSpec(memory_space=pltpu.VMEM),
        scratch_shapes=[
            pltpu.VMEM((m, n), jnp.float32),
            pltpu.VMEM((3, chunk, n2), jnp.float32),
            pltpu.VMEM((3, chunk, n2), jnp.float32),
            pltpu.SemaphoreType.DMA((3,)),
            pltpu.SemaphoreType.DMA((3,)),
            pltpu.SemaphoreType.DMA((3,)),
            pltpu.SemaphoreType.DMA((3,)),
        ],
        compiler_params=pltpu.CompilerParams(collective_id=0),
    )(x, w_mat)


# device time: 25624 ns/iter; 1.8231x vs baseline; 1.0157x over previous
import jax
import jax.numpy as jnp
from jax import lax
from jax.experimental import pallas as pl
from jax.experimental.pallas import tpu as pltpu

N_DEV = 4


def _gelu(y):
    c = 0.7978845608028654
    return 0.5 * y * (1.0 + jnp.tanh(c * (y + 0.044715 * y * y * y)))


def kernel(x, w_mat):
    m, k_per = x.shape
    _, n = w_mat.shape
    chunk = m // N_DEV
    n2 = n // 2

    def body(x_ref, w_ref, out_ref, y_ref, xbuf, ybuf,
             x_send, x_recv, y_send, y_recv):
        my = lax.axis_index("i")
        c_yp = lax.bitwise_xor(my, 1)
        c_xp = 3 - my
        c_diag = lax.rem(my + 2, N_DEV)

        barrier_sem = pltpu.get_barrier_semaphore()
        for nbr in (c_yp, c_xp):
            pl.semaphore_signal(
                barrier_sem, inc=1,
                device_id=(nbr,), device_id_type=pl.DeviceIdType.MESH,
            )

        def rows(c):
            return pl.ds(c * chunk, chunk)

        def compute_chunk(c):
            y_ref[rows(c), :] = jnp.dot(
                x_ref[rows(c), :], w_ref[...],
                preferred_element_type=jnp.float32,
            )

        X = pl.ds(n2, n2)
        Y = pl.ds(0, n2)

        def send(src, buf, slot, ssem, rsem, tgt):
            r = pltpu.make_async_remote_copy(
                src_ref=src,
                dst_ref=buf.at[slot],
                send_sem=ssem.at[slot],
                recv_sem=rsem.at[slot],
                device_id=(tgt,), device_id_type=pl.DeviceIdType.MESH,
            )
            r.start()
            return r

        compute_chunk(c_diag)
        pl.semaphore_wait(barrier_sem, 2)
        x_p1a = send(y_ref.at[rows(c_diag), X], xbuf, 0, x_send, x_recv, c_xp)
        y_p1a = send(y_ref.at[rows(c_diag), Y], ybuf, 0, y_send, y_recv, c_yp)
        compute_chunk(c_yp)
        y_p1b = send(y_ref.at[rows(c_yp), Y], ybuf, 1, y_send, y_recv, c_yp)
        compute_chunk(c_xp)
        x_p1b = send(y_ref.at[rows(c_xp), X], xbuf, 1, x_send, x_recv, c_xp)
        compute_chunk(my)

        x_p1a.wait_recv()
        xbuf[0] = xbuf[0] + y_ref[rows(c_yp), X]
        y_p2 = send(xbuf.at[0], ybuf, 2, y_send, y_recv, c_yp)

        y_p1a.wait_recv()
        ybuf[0] = ybuf[0] + y_ref[rows(c_xp), Y]
        x_p2 = send(ybuf.at[0], xbuf, 2, x_send, x_recv, c_xp)

        x_p1b.wait_recv()
        y_p1b.wait_recv()
        y_p2.wait_recv()
        x_p2.wait_recv()
        out_ref[:, 0:n2] = _gelu(y_ref[rows(my), Y] + ybuf[1] + xbuf[2])
        out_ref[:, n2:n] = _gelu(y_ref[rows(my), X] + xbuf[1] + ybuf[2])

        for r in (x_p1a, y_p1a, y_p1b, x_p1b, y_p2, x_p2):
            r.wait_send()

    return pl.pallas_call(
        body,
        out_shape=jax.ShapeDtypeStruct((chunk, n), jnp.float32),
        in_specs=[
            pl.BlockSpec(memory_space=pltpu.VMEM),
            pl.BlockSpec(memory_space=pltpu.VMEM),
        ],
        out_specs=pl.BlockSpec(memory_space=pltpu.VMEM),
        scratch_shapes=[
            pltpu.VMEM((m, n), jnp.float32),
            pltpu.VMEM((3, chunk, n2), jnp.float32),
            pltpu.VMEM((3, chunk, n2), jnp.float32),
            pltpu.SemaphoreType.DMA((3,)),
            pltpu.SemaphoreType.DMA((3,)),
            pltpu.SemaphoreType.DMA((3,)),
            pltpu.SemaphoreType.DMA((3,)),
        ],
        compiler_params=pltpu.CompilerParams(collective_id=0),
    )(x, w_mat)


# device time: 17253 ns/iter; 2.7076x vs baseline; 1.4852x over previous
import jax
import jax.numpy as jnp
from jax import lax
from jax.experimental import pallas as pl
from jax.experimental.pallas import tpu as pltpu

N_DEV = 4


def _gelu(y):
    c = 0.7978845608028654
    return 0.5 * y * (1.0 + jnp.tanh(c * (y + 0.044715 * y * y * y)))


def kernel(x, w_mat):
    m, k_per = x.shape
    _, n = w_mat.shape
    chunk = m // N_DEV
    n2 = n // 2
    bf16 = jnp.bfloat16

    def body(x_ref, w_ref, out_ref, y_ref, sbuf, xbuf, ybuf,
             x_send, x_recv, y_send, y_recv):
        my = lax.axis_index("i")
        c_yp = lax.bitwise_xor(my, 1)
        c_xp = 3 - my
        c_diag = lax.rem(my + 2, N_DEV)

        barrier_sem = pltpu.get_barrier_semaphore()
        for nbr in (c_yp, c_xp):
            pl.semaphore_signal(
                barrier_sem, inc=1,
                device_id=(nbr,), device_id_type=pl.DeviceIdType.MESH,
            )

        def rows(c):
            return pl.ds(c * chunk, chunk)

        def compute_chunk(c):
            y_ref[rows(c), :] = jnp.dot(
                x_ref[rows(c), :], w_ref[...],
                preferred_element_type=jnp.float32,
            )

        X = pl.ds(n2, n2)
        Y = pl.ds(0, n2)

        S_XA, S_XB, S_YA, S_YB, S_P2X, S_P2Y = range(6)

        def send(src, dst, ssem, rsem, tgt):
            r = pltpu.make_async_remote_copy(
                src_ref=src, dst_ref=dst, send_sem=ssem, recv_sem=rsem,
                device_id=(tgt,), device_id_type=pl.DeviceIdType.MESH,
            )
            r.start()
            return r

        y_ref[rows(c_diag), n2:n] = jnp.dot(
            x_ref[rows(c_diag), :], w_ref[:, n2:n],
            preferred_element_type=jnp.float32,
        )
        sbuf[S_XA] = y_ref[rows(c_diag), X].astype(bf16)
        pl.semaphore_wait(barrier_sem, 2)
        x_p1a = send(sbuf.at[S_XA], xbuf.at[0],
                     x_send.at[0], x_recv.at[0], c_xp)
        y_ref[rows(c_diag), 0:n2] = jnp.dot(
            x_ref[rows(c_diag), :], w_ref[:, 0:n2],
            preferred_element_type=jnp.float32,
        )
        sbuf[S_YA] = y_ref[rows(c_diag), Y].astype(bf16)
        y_p1a = send(sbuf.at[S_YA], ybuf.at[0],
                     y_send.at[0], y_recv.at[0], c_yp)
        compute_chunk(c_yp)
        sbuf[S_YB] = y_ref[rows(c_yp), Y].astype(bf16)
        y_p1b = send(sbuf.at[S_YB], ybuf.at[1],
                     y_send.at[1], y_recv.at[1], c_yp)
        compute_chunk(c_xp)
        sbuf[S_XB] = y_ref[rows(c_xp), X].astype(bf16)
        x_p1b = send(sbuf.at[S_XB], xbuf.at[1],
                     x_send.at[1], x_recv.at[1], c_xp)
        compute_chunk(my)

        nq = n2 // 2

        def qsl(k):
            return pl.ds(k * nq, nq)

        x_p1a.wait_recv()
        sbuf[S_P2Y] = (
            xbuf[0].astype(jnp.float32) + y_ref[rows(c_yp), X]
        ).astype(bf16)
        y_p2 = [
            send(sbuf.at[S_P2Y, :, qsl(k)], ybuf.at[2, :, qsl(k)],
                 y_send.at[2 + k], y_recv.at[2 + k], c_yp)
            for k in range(2)
        ]

        y_p1a.wait_recv()
        sbuf[S_P2X] = (
            ybuf[0].astype(jnp.float32) + y_ref[rows(c_xp), Y]
        ).astype(bf16)
        x_p2 = [
            send(sbuf.at[S_P2X, :, qsl(k)], xbuf.at[2, :, qsl(k)],
                 x_send.at[2 + k], x_recv.at[2 + k], c_xp)
            for k in range(2)
        ]

        y_p1b.wait_recv()
        for k in range(2):
            x_p2[k].wait_recv()
            out_ref[:, k * nq:(k + 1) * nq] = _gelu(
                y_ref[rows(my), pl.ds(k * nq, nq)]
                + ybuf[1, :, qsl(k)].astype(jnp.float32)
                + xbuf[2, :, qsl(k)].astype(jnp.float32)
            )
        x_p1b.wait_recv()
        for k in range(2):
            y_p2[k].wait_recv()
            out_ref[:, n2 + k * nq:n2 + (k + 1) * nq] = _gelu(
                y_ref[rows(my), pl.ds(n2 + k * nq, nq)]
                + xbuf[1, :, qsl(k)].astype(jnp.float32)
                + ybuf[2, :, qsl(k)].astype(jnp.float32)
            )

        for r in (x_p1a, y_p1a, y_p1b, x_p1b, *y_p2, *x_p2):
            r.wait_send()

    return pl.pallas_call(
        body,
        out_shape=jax.ShapeDtypeStruct((chunk, n), jnp.float32),
        in_specs=[
            pl.BlockSpec(memory_space=pltpu.VMEM),
            pl.BlockSpec(memory_space=pltpu.VMEM),
        ],
        out_specs=pl.BlockSpec(memory_space=pltpu.VMEM),
        scratch_shapes=[
            pltpu.VMEM((m, n), jnp.float32),
            pltpu.VMEM((6, chunk, n2), bf16),
            pltpu.VMEM((3, chunk, n2), bf16),
            pltpu.VMEM((3, chunk, n2), bf16),
            pltpu.SemaphoreType.DMA((4,)),
            pltpu.SemaphoreType.DMA((4,)),
            pltpu.SemaphoreType.DMA((4,)),
            pltpu.SemaphoreType.DMA((4,)),
        ],
        compiler_params=pltpu.CompilerParams(collective_id=0),
    )(x, w_mat)
